# baseline (speedup 1.0000x reference)
import math

import jax
import jax.numpy as jnp
from jax import lax
from jax.experimental import pallas as pl
from jax.experimental.pallas import tpu as pltpu
from jax.experimental.pallas import tpu_sc as plsc

D_MODEL = 64
SCALE = math.sqrt(D_MODEL)
NC, NS = 2, 16
NW = NC * NS
CHUNK = 128
NBUF = 4
LOOKAHEAD = 2
L = 16


def _embed_body(xt_hbm, t2_hbm, out_hbm, idxs_v, cbuf, hbufs, bufs, tbufs,
                gsems, ssems):
    wid = lax.axis_index("s") * NC + lax.axis_index("c")
    seq = xt_hbm.shape[0]
    base = wid * CHUNK

    pltpu.sync_copy(xt_hbm.at[:, pl.ds(base, CHUNK)], idxs_v)

    def prefetch(j, b):
        for t in range(CHUNK // L):
            v = idxs_v[j, pl.ds(t * L, L)]
            hbufs[b][pl.ds(t * L, L)] = lax.shift_right_logical(v, 1)
        pltpu.async_copy(t2_hbm.at[hbufs[b]], bufs[b], gsems[b])

    def wait_gather(b):
        pltpu.make_async_copy(t2_hbm.at[hbufs[b]], bufs[b], gsems[b]).wait()

    def start_scatter(j, b):
        pltpu.async_copy(
            tbufs[b], out_hbm.at[j, :, pl.ds(base, CHUNK)], ssems[b])

    def wait_scatter(j, b):
        pltpu.make_async_copy(
            tbufs[b], out_hbm.at[j, :, pl.ds(base, CHUNK)], ssems[b]).wait()

    iota = lax.iota(jnp.int32, L)

    def process(j, b):
        for t in range(CHUNK // L):
            v = idxs_v[j, pl.ds(t * L, L)]
            cbuf[pl.ds(t * L, L)] = (v & 1) * D_MODEL
        buf, tbuf = bufs[b], tbufs[b]

        def fblock(fb, cur):
            f0 = fb * 4
            for t in range(CHUNK // L):
                rows = t * L + iota
                cols = cbuf[pl.ds(t * L, L)]
                for df in range(4):
                    f = f0 + df
                    vals = plsc.load_gather(buf, [rows, cols + f]) * SCALE
                    tbuf[f, pl.ds(t * L, L)] = vals
            return cur

        lax.fori_loop(0, D_MODEL // 4, fblock, jnp.int32(0))

    for k in range(LOOKAHEAD):
        prefetch(k, k)

    def outer(o, cur):
        for b in range(NBUF):
            j = o * NBUF + b
            pb = (b + LOOKAHEAD) % NBUF
            jp = j + LOOKAHEAD

            @pl.when(jnp.logical_and(jp < seq, jp >= NBUF))
            def _():
                wait_scatter(jp - NBUF, pb)

            @pl.when(jp < seq)
            def _():
                prefetch(jp, pb)

            wait_gather(b)
            process(j, b)
            start_scatter(j, b)
        return cur

    lax.fori_loop(0, seq // NBUF, outer, jnp.int32(0))

    for b in range(NBUF):
        wait_scatter(seq - NBUF + b, b)


def kernel(x, table):
    n_batch, seq = x.shape
    vocab = table.shape[0]
    xt = x.astype(jnp.int32).T
    t2 = table.reshape(vocab // 2, 2 * D_MODEL)
    mesh = plsc.VectorSubcoreMesh(core_axis_name="c", subcore_axis_name="s")

    def body(xt_hbm, t2_hbm, out_hbm, idxs_v, cbuf,
             h0, h1, h2, h3, b0, b1, b2, b3, t0, t1, t2_, t3,
             g0, g1, g2, g3, s0, s1, s2, s3):
        _embed_body(xt_hbm, t2_hbm, out_hbm, idxs_v, cbuf,
                    (h0, h1, h2, h3), (b0, b1, b2, b3), (t0, t1, t2_, t3),
                    (g0, g1, g2, g3), (s0, s1, s2, s3))

    out_t = pl.kernel(
        body,
        out_type=jax.ShapeDtypeStruct((seq, D_MODEL, n_batch), jnp.float32),
        mesh=mesh,
        compiler_params=pltpu.CompilerParams(needs_layout_passes=False),
        scratch_types=[
            pltpu.VMEM((seq, CHUNK), jnp.int32),
            pltpu.VMEM((CHUNK,), jnp.int32),
        ] + [pltpu.VMEM((CHUNK,), jnp.int32)] * NBUF
          + [pltpu.VMEM((CHUNK, 2 * D_MODEL), jnp.float32)] * NBUF
          + [pltpu.VMEM((D_MODEL, CHUNK), jnp.float32)] * NBUF
          + [pltpu.SemaphoreType.DMA] * (2 * NBUF),
    )(xt, t2)
    return out_t.transpose(2, 0, 1)


# speedup vs baseline: 1.6065x; 1.6065x over previous
import math

import jax
import jax.numpy as jnp
from jax import lax
from jax.experimental import pallas as pl
from jax.experimental.pallas import tpu as pltpu
from jax.experimental.pallas import tpu_sc as plsc

D_MODEL = 64
SCALE = math.sqrt(D_MODEL)
NC, NS = 2, 16
NW = NC * NS
CHUNK = 128
NBUF = 4
LOOKAHEAD = 2
L = 16


def _embed_body(xt_hbm, t2_hbm, out_hbm, idxs_v, cbuf, hbufs, bufs, tbufs,
                gsems, ssems):
    wid = lax.axis_index("s") * NC + lax.axis_index("c")
    seq = xt_hbm.shape[0]
    base = wid * CHUNK

    pltpu.sync_copy(xt_hbm.at[:, pl.ds(base, CHUNK)], idxs_v)

    def prefetch(j, b):
        for t in range(CHUNK // L):
            v = idxs_v[j, pl.ds(t * L, L)]
            hbufs[b][pl.ds(t * L, L)] = lax.shift_right_logical(v, 1)
        pltpu.async_copy(t2_hbm.at[hbufs[b]], bufs[b], gsems[b])

    def wait_gather(b):
        pltpu.make_async_copy(t2_hbm.at[hbufs[b]], bufs[b], gsems[b]).wait()

    def start_scatter(j, b):
        pltpu.async_copy(
            tbufs[b], out_hbm.at[j, :, pl.ds(base, CHUNK)], ssems[b])

    def wait_scatter(j, b):
        pltpu.make_async_copy(
            tbufs[b], out_hbm.at[j, :, pl.ds(base, CHUNK)], ssems[b]).wait()

    iota = lax.iota(jnp.int32, L)

    def process(j, b):
        for t in range(CHUNK // L):
            v = idxs_v[j, pl.ds(t * L, L)]
            cbuf[pl.ds(t * L, L)] = (v & 1) * D_MODEL
        buf, tbuf = bufs[b], tbufs[b]

        @plsc.parallel_loop(0, D_MODEL // 4, step=1, unroll=4)
        def _(fb):
            f0 = fb * 4
            for t in range(CHUNK // L):
                rows = t * L + iota
                cols = cbuf[pl.ds(t * L, L)]
                for df in range(4):
                    f = f0 + df
                    vals = plsc.load_gather(buf, [rows, cols + f]) * SCALE
                    tbuf[f, pl.ds(t * L, L)] = vals

    for k in range(LOOKAHEAD):
        prefetch(k, k)

    def outer(o, cur):
        for b in range(NBUF):
            j = o * NBUF + b
            pb = (b + LOOKAHEAD) % NBUF
            jp = j + LOOKAHEAD

            @pl.when(jnp.logical_and(jp < seq, jp >= NBUF))
            def _():
                wait_scatter(jp - NBUF, pb)

            @pl.when(jp < seq)
            def _():
                prefetch(jp, pb)

            wait_gather(b)
            process(j, b)
            start_scatter(j, b)
        return cur

    lax.fori_loop(0, seq // NBUF, outer, jnp.int32(0))

    for b in range(NBUF):
        wait_scatter(seq - NBUF + b, b)


def kernel(x, table):
    n_batch, seq = x.shape
    vocab = table.shape[0]
    xt = x.astype(jnp.int32).T
    t2 = table.reshape(vocab // 2, 2 * D_MODEL)
    mesh = plsc.VectorSubcoreMesh(core_axis_name="c", subcore_axis_name="s")

    def body(xt_hbm, t2_hbm, out_hbm, idxs_v, cbuf,
             h0, h1, h2, h3, b0, b1, b2, b3, t0, t1, t2_, t3,
             g0, g1, g2, g3, s0, s1, s2, s3):
        _embed_body(xt_hbm, t2_hbm, out_hbm, idxs_v, cbuf,
                    (h0, h1, h2, h3), (b0, b1, b2, b3), (t0, t1, t2_, t3),
                    (g0, g1, g2, g3), (s0, s1, s2, s3))

    out_t = pl.kernel(
        body,
        out_type=jax.ShapeDtypeStruct((seq, D_MODEL, n_batch), jnp.float32),
        mesh=mesh,
        compiler_params=pltpu.CompilerParams(needs_layout_passes=False),
        scratch_types=[
            pltpu.VMEM((seq, CHUNK), jnp.int32),
            pltpu.VMEM((CHUNK,), jnp.int32),
        ] + [pltpu.VMEM((CHUNK,), jnp.int32)] * NBUF
          + [pltpu.VMEM((CHUNK, 2 * D_MODEL), jnp.float32)] * NBUF
          + [pltpu.VMEM((D_MODEL, CHUNK), jnp.float32)] * NBUF
          + [pltpu.SemaphoreType.DMA] * (2 * NBUF),
    )(xt, t2)
    return out_t.transpose(2, 0, 1)


# speedup vs baseline: 1.7210x; 1.0713x over previous
import math

import jax
import jax.numpy as jnp
from jax import lax
from jax.experimental import pallas as pl
from jax.experimental.pallas import tpu as pltpu
from jax.experimental.pallas import tpu_sc as plsc

D_MODEL = 64
SCALE = math.sqrt(D_MODEL)
NC, NS = 2, 16
NW = NC * NS
CHUNK = 128
NBUF = 4
LOOKAHEAD = 2


def _scale_chunk(buf):
    @plsc.parallel_loop(0, CHUNK, step=1, unroll=8)
    def _(i):
        for j in range(D_MODEL // 16):
            buf[i, pl.ds(j * 16, 16)] = buf[i, pl.ds(j * 16, 16)] * SCALE


def _embed_body(x_hbm, table_hbm, out_hbm, idx_v, bufs, gsems, ssems):
    wid = lax.axis_index("s") * NC + lax.axis_index("c")
    per_w = x_hbm.shape[0] // NW
    nchunk = per_w // CHUNK
    base = wid * per_w

    pltpu.sync_copy(x_hbm.at[pl.ds(base, per_w)], idx_v)

    def start_gather(g, b):
        pltpu.async_copy(
            table_hbm.at[idx_v.at[pl.ds(g * CHUNK, CHUNK)]], bufs[b], gsems[b])

    def wait_gather(g, b):
        pltpu.make_async_copy(
            table_hbm.at[idx_v.at[pl.ds(g * CHUNK, CHUNK)]], bufs[b], gsems[b]).wait()

    def start_scatter(g, b):
        pltpu.async_copy(
            bufs[b], out_hbm.at[pl.ds(base + g * CHUNK, CHUNK)], ssems[b])

    def wait_scatter(g, b):
        pltpu.make_async_copy(
            bufs[b], out_hbm.at[pl.ds(base + g * CHUNK, CHUNK)], ssems[b]).wait()

    for k in range(LOOKAHEAD):
        start_gather(k, k)

    def outer(o, cur):
        for b in range(NBUF):
            g = o * NBUF + b
            pb = (b + LOOKAHEAD) % NBUF
            gp = g + LOOKAHEAD

            @pl.when(jnp.logical_and(gp < nchunk, gp >= NBUF))
            def _():
                wait_scatter(gp - NBUF, pb)

            @pl.when(gp < nchunk)
            def _():
                start_gather(gp, pb)

            wait_gather(g, b)
            _scale_chunk(bufs[b])
            start_scatter(g, b)
        return cur

    lax.fori_loop(0, nchunk // NBUF, outer, jnp.int32(0))

    for b in range(NBUF):
        wait_scatter(nchunk - NBUF + b, b)


def kernel(x, table):
    x_flat = x.reshape(-1).astype(jnp.int32)
    b_total = x_flat.shape[0]
    mesh = plsc.VectorSubcoreMesh(core_axis_name="c", subcore_axis_name="s")
    per_w = b_total // NW

    def body(x_hbm, table_hbm, out_hbm,
             idx_v, b0, b1, b2, b3, g0, g1, g2, g3, s0, s1, s2, s3):
        _embed_body(x_hbm, table_hbm, out_hbm, idx_v,
                    (b0, b1, b2, b3), (g0, g1, g2, g3), (s0, s1, s2, s3))

    out = pl.kernel(
        body,
        out_type=jax.ShapeDtypeStruct((b_total, D_MODEL), jnp.float32),
        mesh=mesh,
        compiler_params=pltpu.CompilerParams(use_tc_tiling_on_sc=False),
        scratch_types=[
            pltpu.VMEM((per_w,), jnp.int32),
        ] + [pltpu.VMEM((CHUNK, D_MODEL), jnp.float32)] * NBUF
          + [pltpu.SemaphoreType.DMA] * (2 * NBUF),
    )(x_flat, table)
    return out.reshape(x.shape + (D_MODEL,))


# speedup vs baseline: 4.1514x; 2.4123x over previous
import math

import jax
import jax.numpy as jnp
from jax import lax
from jax.experimental import pallas as pl
from jax.experimental.pallas import tpu as pltpu
from jax.experimental.pallas import tpu_sc as plsc

D_MODEL = 64
SCALE = math.sqrt(D_MODEL)
NC, NS = 2, 16
NW = NC * NS
CHUNK = 128
NBUF = 4
LOOKAHEAD = 2
L = 16


def _tformat_body(tt_hbm, tail_hbm, wide_hbm, in0, in1, tb0, tb1,
                  gs0, gs1, ss0, ss1):
    wid = lax.axis_index("s") * NC + lax.axis_index("c")
    vocab = tt_hbm.shape[1]
    nch = vocab // CHUNK
    tail = vocab - nch * CHUNK
    total = nch + (1 if tail else 0)
    per = total // NW
    extra = total - per * NW
    mine = per + jnp.where(wid < extra, 1, 0)
    ins, tbs, gss, sss = (in0, in1), (tb0, tb1), (gs0, gs1), (ss0, ss1)
    iota = lax.iota(jnp.int32, L)
    last = total - 1

    def c0_of(c):
        return jnp.where(c == last, vocab - CHUNK, c * CHUNK) if tail else c * CHUNK

    def start(k, b):
        c = k * NW + wid
        if tail:
            @pl.when(c == last)
            def _():
                pltpu.async_copy(tail_hbm, ins[b], gss[b])

            @pl.when(c != last)
            def _():
                pltpu.async_copy(
                    tt_hbm.at[:, pl.ds(c * CHUNK, CHUNK)], ins[b], gss[b])
        else:
            pltpu.async_copy(
                tt_hbm.at[:, pl.ds(c * CHUNK, CHUNK)], ins[b], gss[b])

    def drain(k, b):
        c = k * NW + wid
        c0 = c0_of(c)
        pltpu.make_async_copy(tail_hbm, ins[b], gss[b]).wait()
        src, tbuf = ins[b], tbs[b]

        @plsc.parallel_loop(0, D_MODEL, step=1, unroll=8)
        def _(f):
            fvec = (f + iota) & (D_MODEL - 1)
            for t in range(CHUNK // L):
                toks = t * L + iota
                vals = plsc.load_gather(src, [fvec, toks]) * SCALE
                plsc.store_scatter(tbuf, [toks >> 1, (toks & 1) * D_MODEL + fvec],
                                   vals)

        row0 = pl.multiple_of(c0 // 2, 8)
        pltpu.async_copy(
            tbuf, wide_hbm.at[pl.ds(row0, CHUNK // 2), :], sss[b])

    def wait_store(b):
        pltpu.make_async_copy(
            tbs[b], wide_hbm.at[pl.ds(0, CHUNK // 2), :], sss[b]).wait()

    @pl.when(mine > 0)
    def _():
        start(0, 0)

    def step(k, cur):
        for b in range(2):
            @pl.when(jnp.logical_and(lax.rem(k, 2) == b, k < mine))
            def _():
                @pl.when(k + 1 < mine)
                def _():
                    start(k + 1, 1 - b)

                @pl.when(k >= 2)
                def _():
                    wait_store(b)
                drain(k, b)
        return cur

    lax.fori_loop(0, per + 1, step, jnp.int32(0))

    for b in range(2):
        @pl.when(mine > b)
        def _():
            wait_store(b)


def _gather_body(xt_hbm, wide_hbm, out_hbm, idxs_v, cbuf,
                 hbufs, bufs, tbufs, gsems, ssems):
    wid = lax.axis_index("s") * NC + lax.axis_index("c")
    seq = xt_hbm.shape[0]
    base = wid * CHUNK
    iota = lax.iota(jnp.int32, L)

    pltpu.sync_copy(xt_hbm.at[:, pl.ds(base, CHUNK)], idxs_v)

    def start_gather(j, b):
        for t in range(CHUNK // L):
            v = idxs_v[j, pl.ds(t * L, L)]
            hbufs[b][pl.ds(t * L, L)] = lax.shift_right_logical(v, 1)
        pltpu.async_copy(wide_hbm.at[hbufs[b]], bufs[b], gsems[b])

    def wait_gather(b):
        pltpu.make_async_copy(wide_hbm.at[hbufs[b]], bufs[b], gsems[b]).wait()

    def start_scatter(j, b):
        pltpu.async_copy(tbufs[b], out_hbm.at[j, :, pl.ds(base, CHUNK)], ssems[b])

    def wait_scatter(j, b):
        pltpu.make_async_copy(
            tbufs[b], out_hbm.at[j, :, pl.ds(base, CHUNK)], ssems[b]).wait()

    def process(j, b):
        for t in range(CHUNK // L):
            v = idxs_v[j, pl.ds(t * L, L)]
            cbuf[pl.ds(t * L, L)] = (v & 1) * D_MODEL
        buf, tbuf = bufs[b], tbufs[b]

        @plsc.parallel_loop(0, D_MODEL, step=1, unroll=8)
        def _(f):
            fvec = (f + iota) & (D_MODEL - 1)
            for t in range(CHUNK // L):
                rows = t * L + iota
                cols = cbuf[pl.ds(t * L, L)] + fvec
                vals = plsc.load_gather(buf, [rows, cols])
                plsc.store_scatter(tbuf, [fvec, rows], vals)

    for k in range(LOOKAHEAD):
        start_gather(k, k)

    def outer(o, cur):
        for b in range(NBUF):
            j = o * NBUF + b
            pb = (b + LOOKAHEAD) % NBUF
            jp = j + LOOKAHEAD

            @pl.when(jnp.logical_and(jp < seq, jp >= NBUF))
            def _():
                wait_scatter(jp - NBUF, pb)

            @pl.when(jp < seq)
            def _():
                start_gather(jp, pb)

            wait_gather(b)
            process(j, b)
            start_scatter(j, b)
        return cur

    lax.fori_loop(0, seq // NBUF, outer, jnp.int32(0))

    for b in range(NBUF):
        wait_scatter(seq - NBUF + b, b)


def kernel(x, table):
    n_batch, seq = x.shape
    vocab = table.shape[0]
    xt = x.astype(jnp.int32).T
    tt = table.T
    tail_t = table[vocab - CHUNK:].T
    mesh = plsc.VectorSubcoreMesh(core_axis_name="c", subcore_axis_name="s")
    params = pltpu.CompilerParams(needs_layout_passes=False)

    wide = pl.kernel(
        _tformat_body,
        out_type=jax.ShapeDtypeStruct((vocab // 2, 2 * D_MODEL), jnp.float32),
        mesh=mesh,
        compiler_params=params,
        scratch_types=[pltpu.VMEM((D_MODEL, CHUNK), jnp.float32)] * 2
                      + [pltpu.VMEM((CHUNK // 2, 2 * D_MODEL), jnp.float32)] * 2
                      + [pltpu.SemaphoreType.DMA] * 4,
    )(tt, tail_t)

    def body(xt_hbm, wide_hbm, out_hbm, idxs_v, cbuf,
             h0, h1, h2, h3, b0, b1, b2, b3, t0, t1, t2_, t3,
             g0, g1, g2, g3, s0, s1, s2, s3):
        _gather_body(xt_hbm, wide_hbm, out_hbm, idxs_v, cbuf,
                     (h0, h1, h2, h3), (b0, b1, b2, b3), (t0, t1, t2_, t3),
                     (g0, g1, g2, g3), (s0, s1, s2, s3))

    out_t = pl.kernel(
        body,
        out_type=jax.ShapeDtypeStruct((seq, D_MODEL, n_batch), jnp.float32),
        mesh=mesh,
        compiler_params=params,
        scratch_types=[
            pltpu.VMEM((seq, CHUNK), jnp.int32),
            pltpu.VMEM((CHUNK,), jnp.int32),
        ] + [pltpu.VMEM((CHUNK,), jnp.int32)] * NBUF
          + [pltpu.VMEM((CHUNK, 2 * D_MODEL), jnp.float32)] * NBUF
          + [pltpu.VMEM((D_MODEL, CHUNK), jnp.float32)] * NBUF
          + [pltpu.SemaphoreType.DMA] * (2 * NBUF),
    )(xt, wide)
    return out_t.transpose(2, 0, 1)


# speedup vs baseline: 4.5627x; 1.0991x over previous
import math

import jax
import jax.numpy as jnp
from jax import lax
from jax.experimental import pallas as pl
from jax.experimental.pallas import tpu as pltpu
from jax.experimental.pallas import tpu_sc as plsc

D_MODEL = 64
SCALE = math.sqrt(D_MODEL)
NC, NS = 2, 16
NW = NC * NS
CHUNK = 128
NBUF = 4
LOOKAHEAD = 2
L = 16


def _tformat_body(tt_hbm, tail_hbm, wide_hbm, in0, in1, tb0, tb1,
                  gs0, gs1, ss0, ss1):
    wid = lax.axis_index("s") * NC + lax.axis_index("c")
    vocab = tt_hbm.shape[1]
    nch = vocab // CHUNK
    tail = vocab - nch * CHUNK
    total = nch + (1 if tail else 0)
    per = total // NW
    extra = total - per * NW
    mine = per + jnp.where(wid < extra, 1, 0)
    ins, tbs, gss, sss = (in0, in1), (tb0, tb1), (gs0, gs1), (ss0, ss1)
    iota = lax.iota(jnp.int32, L)
    last = total - 1

    def c0_of(c):
        return jnp.where(c == last, vocab - CHUNK, c * CHUNK) if tail else c * CHUNK

    def start(k, b):
        c = k * NW + wid
        if tail:
            @pl.when(c == last)
            def _():
                pltpu.async_copy(tail_hbm, ins[b], gss[b])

            @pl.when(c != last)
            def _():
                pltpu.async_copy(
                    tt_hbm.at[:, pl.ds(c * CHUNK, CHUNK)], ins[b], gss[b])
        else:
            pltpu.async_copy(
                tt_hbm.at[:, pl.ds(c * CHUNK, CHUNK)], ins[b], gss[b])

    def drain(k, b):
        c = k * NW + wid
        c0 = c0_of(c)
        pltpu.make_async_copy(tail_hbm, ins[b], gss[b]).wait()
        src, tbuf = ins[b], tbs[b]

        @plsc.parallel_loop(0, D_MODEL, step=1, unroll=8)
        def _(f):
            fvec = (f + iota) & (D_MODEL - 1)
            for t in range(CHUNK // L):
                toks = t * L + iota
                vals = plsc.load_gather(src, [fvec, toks]) * SCALE
                plsc.store_scatter(tbuf, [toks >> 1, (toks & 1) * D_MODEL + fvec],
                                   vals)

        row0 = pl.multiple_of(c0 // 2, 8)
        pltpu.async_copy(
            tbuf, wide_hbm.at[pl.ds(row0, CHUNK // 2), :], sss[b])

    def wait_store(b):
        pltpu.make_async_copy(
            tbs[b], wide_hbm.at[pl.ds(0, CHUNK // 2), :], sss[b]).wait()

    @pl.when(mine > 0)
    def _():
        start(0, 0)

    def step(k, cur):
        for b in range(2):
            @pl.when(jnp.logical_and(lax.rem(k, 2) == b, k < mine))
            def _():
                @pl.when(k + 1 < mine)
                def _():
                    start(k + 1, 1 - b)

                @pl.when(k >= 2)
                def _():
                    wait_store(b)
                drain(k, b)
        return cur

    lax.fori_loop(0, per + 1, step, jnp.int32(0))

    for b in range(2):
        @pl.when(mine > b)
        def _():
            wait_store(b)


def _gather_body(xt_hbm, wide_hbm, out_hbm, idxs_v, cbuf,
                 hbufs, bufs, tbufs, gsems, ssems):
    wid = lax.axis_index("s") * NC + lax.axis_index("c")
    seq = xt_hbm.shape[0]
    base = wid * CHUNK
    iota = lax.iota(jnp.int32, L)

    pltpu.sync_copy(xt_hbm.at[:, pl.ds(base, CHUNK)], idxs_v)

    def start_gather(j, b):
        for t in range(CHUNK // L):
            v = idxs_v[j, pl.ds(t * L, L)]
            hbufs[b][pl.ds(t * L, L)] = lax.shift_right_logical(v, 1)
        pltpu.async_copy(wide_hbm.at[hbufs[b]], bufs[b], gsems[b])

    def wait_gather(b):
        pltpu.make_async_copy(wide_hbm.at[hbufs[b]], bufs[b], gsems[b]).wait()

    def start_scatter(j, b):
        pltpu.async_copy(tbufs[b], out_hbm.at[j, :, pl.ds(base, CHUNK)], ssems[b])

    def wait_scatter(j, b):
        pltpu.make_async_copy(
            tbufs[b], out_hbm.at[j, :, pl.ds(base, CHUNK)], ssems[b]).wait()

    def process(j, b):
        cbs = []
        for t in range(CHUNK // L):
            v = idxs_v[j, pl.ds(t * L, L)]
            cbs.append((v & 1) * D_MODEL)
        buf, tbuf = bufs[b], tbufs[b]

        @plsc.parallel_loop(0, D_MODEL, step=1, unroll=8)
        def _(f):
            fvec = (f + iota) & (D_MODEL - 1)
            for t in range(CHUNK // L):
                rows = t * L + iota
                vals = plsc.load_gather(buf, [rows, cbs[t] + fvec])
                plsc.store_scatter(tbuf, [fvec, rows], vals)

    for k in range(LOOKAHEAD):
        start_gather(k, k)

    def outer(o, cur):
        for b in range(NBUF):
            j = o * NBUF + b
            pb = (b + LOOKAHEAD) % NBUF
            jp = j + LOOKAHEAD

            @pl.when(jnp.logical_and(jp < seq, jp >= NBUF))
            def _():
                wait_scatter(jp - NBUF, pb)

            @pl.when(jp < seq)
            def _():
                start_gather(jp, pb)

            wait_gather(b)
            process(j, b)
            start_scatter(j, b)
        return cur

    lax.fori_loop(0, seq // NBUF, outer, jnp.int32(0))

    for b in range(NBUF):
        wait_scatter(seq - NBUF + b, b)


def kernel(x, table):
    n_batch, seq = x.shape
    vocab = table.shape[0]
    xt = x.astype(jnp.int32).T
    tt = table.T
    tail_t = table[vocab - CHUNK:].T
    mesh = plsc.VectorSubcoreMesh(core_axis_name="c", subcore_axis_name="s")
    params = pltpu.CompilerParams(needs_layout_passes=False)

    wide = pl.kernel(
        _tformat_body,
        out_type=jax.ShapeDtypeStruct((vocab // 2, 2 * D_MODEL), jnp.float32),
        mesh=mesh,
        compiler_params=params,
        scratch_types=[pltpu.VMEM((D_MODEL, CHUNK), jnp.float32)] * 2
                      + [pltpu.VMEM((CHUNK // 2, 2 * D_MODEL), jnp.float32)] * 2
                      + [pltpu.SemaphoreType.DMA] * 4,
    )(tt, tail_t)

    def body(xt_hbm, wide_hbm, out_hbm, idxs_v, cbuf,
             h0, h1, h2, h3, b0, b1, b2, b3, t0, t1, t2_, t3,
             g0, g1, g2, g3, s0, s1, s2, s3):
        _gather_body(xt_hbm, wide_hbm, out_hbm, idxs_v, cbuf,
                     (h0, h1, h2, h3), (b0, b1, b2, b3), (t0, t1, t2_, t3),
                     (g0, g1, g2, g3), (s0, s1, s2, s3))

    out_t = pl.kernel(
        body,
        out_type=jax.ShapeDtypeStruct((seq, D_MODEL, n_batch), jnp.float32),
        mesh=mesh,
        compiler_params=params,
        scratch_types=[
            pltpu.VMEM((seq, CHUNK), jnp.int32),
            pltpu.VMEM((CHUNK,), jnp.int32),
        ] + [pltpu.VMEM((CHUNK,), jnp.int32)] * NBUF
          + [pltpu.VMEM((CHUNK, 2 * D_MODEL), jnp.float32)] * NBUF
          + [pltpu.VMEM((D_MODEL, CHUNK), jnp.float32)] * NBUF
          + [pltpu.SemaphoreType.DMA] * (2 * NBUF),
    )(xt, wide)
    return out_t.transpose(2, 0, 1)


# speedup vs baseline: 4.5908x; 1.0062x over previous
import math

import jax
import jax.numpy as jnp
from jax import lax
from jax.experimental import pallas as pl
from jax.experimental.pallas import tpu as pltpu
from jax.experimental.pallas import tpu_sc as plsc

D_MODEL = 64
SCALE = math.sqrt(D_MODEL)
NC, NS = 2, 16
NW = NC * NS
CHUNK = 128
NBUF = 4
LOOKAHEAD = 2
L = 16


def _tformat_body(tt_hbm, tail_hbm, wide_hbm, in0, in1, tb0, tb1,
                  gs0, gs1, ss0, ss1):
    wid = lax.axis_index("s") * NC + lax.axis_index("c")
    vocab = tt_hbm.shape[1]
    nch = vocab // CHUNK
    tail = vocab - nch * CHUNK
    total = nch + (1 if tail else 0)
    per = total // NW
    extra = total - per * NW
    mine = per + jnp.where(wid < extra, 1, 0)
    ins, tbs, gss, sss = (in0, in1), (tb0, tb1), (gs0, gs1), (ss0, ss1)
    iota = lax.iota(jnp.int32, L)
    last = total - 1

    def c0_of(c):
        return jnp.where(c == last, vocab - CHUNK, c * CHUNK) if tail else c * CHUNK

    def start(k, b):
        c = k * NW + wid
        if tail:
            @pl.when(c == last)
            def _():
                pltpu.async_copy(tail_hbm, ins[b], gss[b])

            @pl.when(c != last)
            def _():
                pltpu.async_copy(
                    tt_hbm.at[:, pl.ds(c * CHUNK, CHUNK)], ins[b], gss[b])
        else:
            pltpu.async_copy(
                tt_hbm.at[:, pl.ds(c * CHUNK, CHUNK)], ins[b], gss[b])

    def drain(k, b):
        c = k * NW + wid
        c0 = c0_of(c)
        pltpu.make_async_copy(tail_hbm, ins[b], gss[b]).wait()
        src, tbuf = ins[b], tbs[b]

        @plsc.parallel_loop(0, D_MODEL, step=1, unroll=16)
        def _(f):
            fvec = (f + iota) & (D_MODEL - 1)
            for t in range(CHUNK // L):
                toks = t * L + iota
                vals = plsc.load_gather(src, [fvec, toks]) * SCALE
                plsc.store_scatter(tbuf, [toks >> 1, (toks & 1) * D_MODEL + fvec],
                                   vals)

        row0 = pl.multiple_of(c0 // 2, 8)
        pltpu.async_copy(
            tbuf, wide_hbm.at[pl.ds(row0, CHUNK // 2), :], sss[b])

    def wait_store(b):
        pltpu.make_async_copy(
            tbs[b], wide_hbm.at[pl.ds(0, CHUNK // 2), :], sss[b]).wait()

    @pl.when(mine > 0)
    def _():
        start(0, 0)

    def step(k, cur):
        for b in range(2):
            @pl.when(jnp.logical_and(lax.rem(k, 2) == b, k < mine))
            def _():
                @pl.when(k + 1 < mine)
                def _():
                    start(k + 1, 1 - b)

                @pl.when(k >= 2)
                def _():
                    wait_store(b)
                drain(k, b)
        return cur

    lax.fori_loop(0, per + 1, step, jnp.int32(0))

    for b in range(2):
        @pl.when(mine > b)
        def _():
            wait_store(b)


def _gather_body(xt_hbm, wide_hbm, out_hbm, idxs_v, cbuf,
                 hbufs, bufs, tbufs, gsems, ssems):
    wid = lax.axis_index("s") * NC + lax.axis_index("c")
    seq = xt_hbm.shape[0]
    base = wid * CHUNK
    iota = lax.iota(jnp.int32, L)

    pltpu.sync_copy(xt_hbm.at[:, pl.ds(base, CHUNK)], idxs_v)

    def start_gather(j, b):
        for t in range(CHUNK // L):
            v = idxs_v[j, pl.ds(t * L, L)]
            hbufs[b][pl.ds(t * L, L)] = lax.shift_right_logical(v, 1)
        pltpu.async_copy(wide_hbm.at[hbufs[b]], bufs[b], gsems[b])

    def wait_gather(b):
        pltpu.make_async_copy(wide_hbm.at[hbufs[b]], bufs[b], gsems[b]).wait()

    def start_scatter(j, b):
        pltpu.async_copy(tbufs[b], out_hbm.at[j, :, pl.ds(base, CHUNK)], ssems[b])

    def wait_scatter(j, b):
        pltpu.make_async_copy(
            tbufs[b], out_hbm.at[j, :, pl.ds(base, CHUNK)], ssems[b]).wait()

    def process(j, b):
        cbs = []
        for t in range(CHUNK // L):
            v = idxs_v[j, pl.ds(t * L, L)]
            cbs.append((v & 1) * D_MODEL)
        buf, tbuf = bufs[b], tbufs[b]

        @plsc.parallel_loop(0, D_MODEL, step=1, unroll=16)
        def _(f):
            fvec = (f + iota) & (D_MODEL - 1)
            for t in range(CHUNK // L):
                rows = t * L + iota
                vals = plsc.load_gather(buf, [rows, cbs[t] + fvec])
                plsc.store_scatter(tbuf, [fvec, rows], vals)

    for k in range(LOOKAHEAD):
        start_gather(k, k)

    def outer(o, cur):
        for b in range(NBUF):
            j = o * NBUF + b
            pb = (b + LOOKAHEAD) % NBUF
            jp = j + LOOKAHEAD

            @pl.when(jnp.logical_and(jp < seq, jp >= NBUF))
            def _():
                wait_scatter(jp - NBUF, pb)

            @pl.when(jp < seq)
            def _():
                start_gather(jp, pb)

            wait_gather(b)
            process(j, b)
            start_scatter(j, b)
        return cur

    lax.fori_loop(0, seq // NBUF, outer, jnp.int32(0))

    for b in range(NBUF):
        wait_scatter(seq - NBUF + b, b)


def kernel(x, table):
    n_batch, seq = x.shape
    vocab = table.shape[0]
    xt = x.astype(jnp.int32).T
    tt = table.T
    tail_t = table[vocab - CHUNK:].T
    mesh = plsc.VectorSubcoreMesh(core_axis_name="c", subcore_axis_name="s")
    params = pltpu.CompilerParams(needs_layout_passes=False)

    wide = pl.kernel(
        _tformat_body,
        out_type=jax.ShapeDtypeStruct((vocab // 2, 2 * D_MODEL), jnp.float32),
        mesh=mesh,
        compiler_params=params,
        scratch_types=[pltpu.VMEM((D_MODEL, CHUNK), jnp.float32)] * 2
                      + [pltpu.VMEM((CHUNK // 2, 2 * D_MODEL), jnp.float32)] * 2
                      + [pltpu.SemaphoreType.DMA] * 4,
    )(tt, tail_t)

    def body(xt_hbm, wide_hbm, out_hbm, idxs_v, cbuf,
             h0, h1, h2, h3, b0, b1, b2, b3, t0, t1, t2_, t3,
             g0, g1, g2, g3, s0, s1, s2, s3):
        _gather_body(xt_hbm, wide_hbm, out_hbm, idxs_v, cbuf,
                     (h0, h1, h2, h3), (b0, b1, b2, b3), (t0, t1, t2_, t3),
                     (g0, g1, g2, g3), (s0, s1, s2, s3))

    out_t = pl.kernel(
        body,
        out_type=jax.ShapeDtypeStruct((seq, D_MODEL, n_batch), jnp.float32),
        mesh=mesh,
        compiler_params=params,
        scratch_types=[
            pltpu.VMEM((seq, CHUNK), jnp.int32),
            pltpu.VMEM((CHUNK,), jnp.int32),
        ] + [pltpu.VMEM((CHUNK,), jnp.int32)] * NBUF
          + [pltpu.VMEM((CHUNK, 2 * D_MODEL), jnp.float32)] * NBUF
          + [pltpu.VMEM((D_MODEL, CHUNK), jnp.float32)] * NBUF
          + [pltpu.SemaphoreType.DMA] * (2 * NBUF),
    )(xt, wide)
    return out_t.transpose(2, 0, 1)
